# baseline (speedup 1.0000x reference)
import jax
import jax.numpy as jnp
from jax import lax
from jax.experimental import pallas as pl
from jax.experimental.pallas import tpu as pltpu
from jax.experimental.pallas import tpu_sc as plsc

BATCH = 4
SEQ_LEN = 4096
D_MODEL = 1024
TOT = BATCH * SEQ_LEN

NUM_CORES = 2
NUM_SUBCORES = 16
NW = NUM_CORES * NUM_SUBCORES
B_PER_W = TOT // NW
W_PER_ROW = SEQ_LEN // B_PER_W

CHUNK = 8
NBUF = 12
NCHUNK = B_PER_W // CHUNK


def _emb_body(ids_hbm, table_hbm, out_hbm, idx_v, rows_v, gsems, ssems):
    wid = lax.axis_index("s") * NUM_CORES + lax.axis_index("c")
    row = wid // W_PER_ROW
    col = pl.multiple_of((wid % W_PER_ROW) * B_PER_W, 8)

    pltpu.sync_copy(ids_hbm.at[row, pl.ds(col, B_PER_W)], idx_v)

    def gather_start(c, b):
        off = pl.multiple_of(c * CHUNK, 8)
        pltpu.async_copy(
            table_hbm.at[idx_v.at[pl.ds(off, CHUNK)]], rows_v.at[b],
            gsems.at[b])

    def gather_wait(b):
        pltpu.make_async_copy(
            table_hbm.at[idx_v.at[pl.ds(0, CHUNK)]], rows_v.at[b],
            gsems.at[b]).wait()

    def store_start(c, b):
        off = pl.multiple_of(col + c * CHUNK, 8)
        pltpu.async_copy(
            rows_v.at[b], out_hbm.at[row, pl.ds(off, CHUNK), :], ssems.at[b])

    def store_wait(b):
        pltpu.make_async_copy(
            rows_v.at[b], out_hbm.at[row, pl.ds(col, CHUNK), :],
            ssems.at[b]).wait()

    for b in range(NBUF):
        gather_start(b, b)

    nfull = (NCHUNK - NBUF) // NBUF

    def group(g, carry):
        for b in range(NBUF):
            c = g * NBUF + b
            gather_wait(b)
            store_start(c, b)
            store_wait(b)
            gather_start(c + NBUF, b)
        return carry

    lax.fori_loop(0, nfull, group, None)

    drained = []
    for c in range(nfull * NBUF, NCHUNK):
        b = c % NBUF
        gather_wait(b)
        store_start(c, b)
        if c + NBUF < NCHUNK:
            store_wait(b)
            gather_start(c + NBUF, b)
        else:
            drained.append(b)
    for b in drained:
        store_wait(b)


@jax.jit
def _emb(ids, table):
    mesh = plsc.VectorSubcoreMesh(
        core_axis_name="c", subcore_axis_name="s",
        num_cores=NUM_CORES, num_subcores=NUM_SUBCORES)
    return pl.kernel(
        _emb_body,
        out_type=jax.ShapeDtypeStruct((BATCH, SEQ_LEN, D_MODEL), jnp.float32),
        mesh=mesh,
        scratch_types=[
            pltpu.VMEM((B_PER_W,), jnp.int32),
            pltpu.VMEM((NBUF, CHUNK, D_MODEL), jnp.float32),
            pltpu.SemaphoreType.DMA((NBUF,)),
            pltpu.SemaphoreType.DMA((NBUF,)),
        ],
    )(ids, table)


def kernel(input_ids, table):
    return _emb(input_ids.astype(jnp.int32), table)


# speedup vs baseline: 1.0186x; 1.0186x over previous
import jax
import jax.numpy as jnp
from jax import lax
from jax.experimental import pallas as pl
from jax.experimental.pallas import tpu as pltpu
from jax.experimental.pallas import tpu_sc as plsc

BATCH = 4
SEQ_LEN = 4096
D_MODEL = 1024
TOT = BATCH * SEQ_LEN

NUM_CORES = 2
NUM_SUBCORES = 16
NW = NUM_CORES * NUM_SUBCORES
B_PER_W = TOT // NW
W_PER_ROW = SEQ_LEN // B_PER_W

CHUNK = 8
NPAIR = 4
NCHUNK = B_PER_W // CHUNK
NPK = NCHUNK // 2


def _emb_body(ids_hbm, table_hbm, out_hbm, idx_v, rows_v, gsems, ssems):
    wid = lax.axis_index("s") * NUM_CORES + lax.axis_index("c")
    row = wid // W_PER_ROW
    col = pl.multiple_of((wid % W_PER_ROW) * B_PER_W, 8)

    pltpu.sync_copy(ids_hbm.at[row, pl.ds(col, B_PER_W)], idx_v)

    def gather_start(c, p, h):
        off = pl.multiple_of(c * CHUNK, 8)
        pltpu.async_copy(
            table_hbm.at[idx_v.at[pl.ds(off, CHUNK)]],
            rows_v.at[p, pl.ds(h * CHUNK, CHUNK), :], gsems.at[2 * p + h])

    def gather_wait(p, h):
        pltpu.make_async_copy(
            table_hbm.at[idx_v.at[pl.ds(0, CHUNK)]],
            rows_v.at[p, pl.ds(h * CHUNK, CHUNK), :],
            gsems.at[2 * p + h]).wait()

    def store_start(k, p):
        off = pl.multiple_of(col + k * 2 * CHUNK, 8)
        pltpu.async_copy(
            rows_v.at[p], out_hbm.at[row, pl.ds(off, 2 * CHUNK), :],
            ssems.at[p])

    def store_wait(p):
        pltpu.make_async_copy(
            rows_v.at[p], out_hbm.at[row, pl.ds(col, 2 * CHUNK), :],
            ssems.at[p]).wait()

    for p in range(NPAIR):
        for h in range(2):
            gather_start(2 * p + h, p, h)

    nfull = NPK - NPAIR

    def pair_step(k, carry):
        p = lax.rem(k, NPAIR)
        gather_wait(p, 0)
        gather_wait(p, 1)
        store_start(k, p)
        store_wait(p)
        kn = k + NPAIR
        gather_start(2 * kn, p, 0)
        gather_start(2 * kn + 1, p, 1)
        return carry

    lax.fori_loop(0, nfull, pair_step, None)

    for k in range(nfull, NPK):
        p = k % NPAIR
        gather_wait(p, 0)
        gather_wait(p, 1)
        store_start(k, p)
    for k in range(nfull, NPK):
        store_wait(k % NPAIR)


@jax.jit
def _emb(ids, table):
    mesh = plsc.VectorSubcoreMesh(
        core_axis_name="c", subcore_axis_name="s",
        num_cores=NUM_CORES, num_subcores=NUM_SUBCORES)
    return pl.kernel(
        _emb_body,
        out_type=jax.ShapeDtypeStruct((BATCH, SEQ_LEN, D_MODEL), jnp.float32),
        mesh=mesh,
        scratch_types=[
            pltpu.VMEM((B_PER_W,), jnp.int32),
            pltpu.VMEM((NPAIR, 2 * CHUNK, D_MODEL), jnp.float32),
            pltpu.SemaphoreType.DMA((2 * NPAIR,)),
            pltpu.SemaphoreType.DMA((NPAIR,)),
        ],
    )(ids, table)


def kernel(input_ids, table):
    return _emb(input_ids.astype(jnp.int32), table)
